# baseline (speedup 1.0000x reference)
import functools

import jax
import jax.numpy as jnp
from jax import lax
from jax.experimental import pallas as pl
from jax.experimental.pallas import tpu as pltpu
from jax.experimental.pallas import tpu_sc as plsc

K_NN = 20
ELL = 0.15
BASEDIST = 10.0
INV_COLOR = 5.0

N = 4096
QT = 8
NQT = N // QT
MASK_IDX = 0xFFF
MASK_VAL = ~0xFFF


G = 16
QTOT = G * QT
NV = N // 128
KNV = K_NN * NV // 128
SUB = 64
INT_MIN = -2147483648
INT_MAX = 0x7FFFFFFF


def _lane128(shape):
    return lax.broadcasted_iota(jnp.int32, shape, 1)


def _xor_perm(x, j, lane):
    return jnp.take_along_axis(x, lane ^ j, axis=1)


def _bitonic128(x):
    lane = _lane128(x.shape)
    for k in (2, 4, 8, 16, 32, 64, 128):
        j = k // 2
        while j >= 1:
            px = _xor_perm(x, j, lane)
            tmin = ((lane & k) == 0) == ((lane & j) == 0)
            x = jnp.where(tmin, jnp.minimum(x, px), jnp.maximum(x, px))
            j //= 2
    return x


def _bitonic_cleanup128(x):
    lane = _lane128(x.shape)
    for j in (64, 32, 16, 8, 4, 2, 1):
        px = _xor_perm(x, j, lane)
        tmin = (lane & j) == 0
        x = jnp.where(tmin, jnp.minimum(x, px), jnp.maximum(x, px))
    return x


def _merge_low128(a, b):
    lane = _lane128(b.shape)
    brev = jnp.take_along_axis(b, 127 - lane, axis=1)
    return _bitonic_cleanup128(jnp.minimum(a, brev))


def _topk_body(q_ref, kt_ref, dk_ref, idx_ref, pk_ref, kb_ref, k2_ref):
    kb_ref[...] = kt_ref[0].astype(jnp.bfloat16)
    k2_ref[...] = jnp.sum(kt_ref[0] * kt_ref[0], axis=0, keepdims=True)

    z = q_ref[0, :, 2]
    ls = jnp.maximum(ELL * (z - BASEDIST) / BASEDIST, ELL)
    neg_inv_ls2 = -1.0 / (ls * ls)

    colmins = []
    for g in range(G):
        qs = pl.ds(g * QT, QT)
        qg = q_ref[0, qs, :]
        qb = qg.astype(jnp.bfloat16)
        qk = jnp.dot(qb, kb_ref[...], preferred_element_type=jnp.float32)
        q2 = jnp.sum(qg * qg, axis=1)
        d2 = (q2[:, None] + k2_ref[...]) - 2.0 * qk
        bits = lax.bitcast_convert_type(d2, jnp.int32)
        sortable = jnp.where(bits >= 0, bits, jnp.int32(INT_MIN) - bits)
        lane = lax.broadcasted_iota(jnp.int32, (QT, N), 1)
        packed = (sortable & jnp.int32(MASK_VAL)) | lane
        pk_ref[qs, :] = packed
        cm = packed[:, :128]
        for v in range(1, NV):
            cm = jnp.minimum(cm, packed[:, v * 128:(v + 1) * 128])
        colmins.append(cm)

    colmin_all = jnp.concatenate(colmins, axis=0)
    for h in range(QTOT // SUB):
        r0 = h * SUB
        colsort = _bitonic128(colmin_all[r0:r0 + SUB])
        idx_cols = colsort[:, :K_NN] & 127
        pk_h = pk_ref[pl.ds(r0, SUB), :]
        cand = jnp.concatenate(
            [jnp.take_along_axis(pk_h[:, v * 128:(v + 1) * 128], idx_cols,
                                 axis=1) for v in range(NV)], axis=1)
        ccm = cand[:, :128]
        for v in range(1, KNV):
            ccm = jnp.minimum(ccm, cand[:, v * 128:(v + 1) * 128])
        clane = _lane128(ccm.shape)
        ccs = _bitonic128((ccm & jnp.int32(~127)) | clane)
        idx_c2 = ccs[:, :K_NN] & 127
        cand2 = jnp.concatenate(
            [jnp.take_along_axis(cand[:, v * 128:(v + 1) * 128], idx_c2,
                                 axis=1) for v in range(KNV)]
            + [jnp.full((SUB, 128 - KNV * K_NN), INT_MAX, jnp.int32)],
            axis=1)
        s = _bitonic128(cand2)
        res = s[:, :K_NN]
        t = res & jnp.int32(MASK_VAL)
        d2t = lax.bitcast_convert_type(
            jnp.where(t >= 0, t, jnp.int32(INT_MIN) - t), jnp.float32)
        dk_ref[0, r0:r0 + SUB] = jnp.exp(
            d2t * neg_inv_ls2[r0:r0 + SUB, None])
        idx_ref[0, r0:r0 + SUB] = res & MASK_IDX


def _topk_call(q_all, kt_all):
    p_count = q_all.shape[0]
    return pl.pallas_call(
        _topk_body,
        grid=(p_count, N // QTOT),
        in_specs=[
            pl.BlockSpec((1, QTOT, 3), lambda p, t: (p, t, 0)),
            pl.BlockSpec((1, 3, N), lambda p, t: (p, 0, 0)),
        ],
        out_specs=[
            pl.BlockSpec((1, QTOT, K_NN), lambda p, t: (p, t, 0)),
            pl.BlockSpec((1, QTOT, K_NN), lambda p, t: (p, t, 0)),
        ],
        out_shape=[
            jax.ShapeDtypeStruct((p_count, N, K_NN), jnp.float32),
            jax.ShapeDtypeStruct((p_count, N, K_NN), jnp.int32),
        ],
        scratch_shapes=[pltpu.VMEM((QTOT, N), jnp.int32),
                        pltpu.VMEM((3, N), jnp.bfloat16),
                        pltpu.VMEM((1, N), jnp.float32)],
    )(q_all, kt_all)


NW = 32
Q_PER_W = (8 * N) // NW
E_PER_W = Q_PER_W * K_NN
LANES = 16
CHUNKS = E_PER_W // LANES


def _sqrt_newton(x):
    b = plsc.bitcast(x, jnp.int32)
    y = plsc.bitcast(jnp.int32(0x1FBD1DF5) + (b >> 1), jnp.float32)
    for _ in range(3):
        y = 0.5 * (y + x / y)
    return y


def _color_body(idx_hbm, dk_hbm, fsrc_hbm, fgt_hbm, out_hbm,
                idx_v, dk_v, fsrc_v, fgt_v, out_v):
    wid = lax.axis_index("s") * 2 + lax.axis_index("c")
    p = wid // 4
    qbase = (wid % 4) * Q_PER_W

    pltpu.sync_copy(idx_hbm.at[wid], idx_v)
    pltpu.sync_copy(dk_hbm.at[wid], dk_v)
    pltpu.sync_copy(fsrc_hbm.at[p], fsrc_v)
    pltpu.sync_copy(fgt_hbm.at[p, pl.ds(qbase, Q_PER_W)], fgt_v)

    lanes = lax.iota(jnp.int32, LANES)

    def body(i, acc):
        base = pl.multiple_of(i * LANES, LANES)
        e = base + lanes
        nn = e // K_NN
        idx16 = idx_v[pl.ds(base, LANES)]
        dk16 = dk_v[pl.ds(base, LANES)]
        cd2 = jnp.zeros((LANES,), jnp.float32)
        for c in range(3):
            cc = jnp.full((LANES,), c, jnp.int32)
            hsv = plsc.load_gather(fsrc_v, [idx16, cc])
            fg = plsc.load_gather(fgt_v, [nn, cc])
            d = fg - hsv
            cd2 = cd2 + d * d
        cd = _sqrt_newton(cd2)
        return acc + dk16 * jnp.exp(cd * (-INV_COLOR))

    acc = lax.fori_loop(0, CHUNKS, body, jnp.zeros((LANES,), jnp.float32))
    out_v[...] = acc
    pltpu.sync_copy(out_v, out_hbm.at[wid])


@functools.partial(jax.jit, static_argnums=())
def _color_call(idx_flat, dk_flat, fsrc_all, fgt_all):
    mesh = plsc.VectorSubcoreMesh(core_axis_name="c", subcore_axis_name="s")
    return pl.kernel(
        _color_body,
        out_type=jax.ShapeDtypeStruct((NW, LANES), jnp.float32),
        mesh=mesh,
        scratch_types=[
            pltpu.VMEM((E_PER_W,), jnp.int32),
            pltpu.VMEM((E_PER_W,), jnp.float32),
            pltpu.VMEM((N, 3), jnp.float32),
            pltpu.VMEM((Q_PER_W, 3), jnp.float32),
            pltpu.VMEM((LANES,), jnp.float32),
        ],
        compiler_params=pltpu.CompilerParams(
            use_tc_tiling_on_sc=False, needs_layout_passes=False),
    )(idx_flat, dk_flat, fsrc_all, fgt_all)


def kernel(pts_gt_1, feat_gt_1, pts_pred_1, feat_pred_1,
           pts_flowed_1_from_2, feat_flowed_1_from_2,
           pts_gt_2, feat_gt_2, pts_pred_2, feat_pred_2,
           pts_flowed_2_from_1, feat_flowed_2_from_1):
    q_all = jnp.concatenate(
        [pts_gt_1, pts_gt_2, pts_gt_1, pts_gt_2], axis=0)
    k_all = jnp.concatenate(
        [pts_pred_1, pts_pred_2, pts_flowed_1_from_2, pts_flowed_2_from_1],
        axis=0)
    fsrc_all = jnp.concatenate(
        [feat_pred_1, feat_pred_2, feat_flowed_1_from_2,
         feat_flowed_2_from_1], axis=0)[:, :, :3]
    fgt_all = jnp.concatenate(
        [feat_gt_1, feat_gt_2, feat_gt_1, feat_gt_2], axis=0)[:, :, :3]

    kt_all = jnp.transpose(k_all, (0, 2, 1))

    dk, idx = _topk_call(q_all, kt_all)
    partial = _color_call(idx.reshape(NW, E_PER_W), dk.reshape(NW, E_PER_W),
                          fsrc_all, fgt_all)
    total = jnp.sum(partial)
    return -(total / jnp.float32(2 * N * K_NN))


# speedup vs baseline: 1.0066x; 1.0066x over previous
import functools

import jax
import jax.numpy as jnp
from jax import lax
from jax.experimental import pallas as pl
from jax.experimental.pallas import tpu as pltpu
from jax.experimental.pallas import tpu_sc as plsc

K_NN = 20
ELL = 0.15
BASEDIST = 10.0
INV_COLOR = 5.0

N = 4096
QT = 8
NQT = N // QT
MASK_IDX = 0xFFF
MASK_VAL = ~0xFFF


G = 16
QTOT = G * QT
NV = N // 128
KNV = K_NN * NV // 128
SUB = 64
INT_MIN = -2147483648
INT_MAX = 0x7FFFFFFF


def _lane128(shape):
    return lax.broadcasted_iota(jnp.int32, shape, 1)


def _xor_perm(x, j, lane):
    return jnp.take_along_axis(x, lane ^ j, axis=1)


def _bitonic128(x):
    lane = _lane128(x.shape)
    for k in (2, 4, 8, 16, 32, 64, 128):
        j = k // 2
        while j >= 1:
            px = _xor_perm(x, j, lane)
            tmin = ((lane & k) == 0) == ((lane & j) == 0)
            x = jnp.where(tmin, jnp.minimum(x, px), jnp.maximum(x, px))
            j //= 2
    return x


def _bitonic_cleanup128(x):
    lane = _lane128(x.shape)
    for j in (64, 32, 16, 8, 4, 2, 1):
        px = _xor_perm(x, j, lane)
        tmin = (lane & j) == 0
        x = jnp.where(tmin, jnp.minimum(x, px), jnp.maximum(x, px))
    return x


def _merge_low128(a, b):
    lane = _lane128(b.shape)
    brev = jnp.take_along_axis(b, 127 - lane, axis=1)
    return _bitonic_cleanup128(jnp.minimum(a, brev))


def _topk_body(q_ref, kt_ref, dk_ref, idx_ref, pk_ref):
    kb = kt_ref[0].astype(jnp.bfloat16)
    k2 = jnp.sum(kt_ref[0] * kt_ref[0], axis=0)

    z = q_ref[0, :, 2]
    ls = jnp.maximum(ELL * (z - BASEDIST) / BASEDIST, ELL)
    neg_inv_ls2 = -1.0 / (ls * ls)

    colmins = []
    for g in range(G):
        qs = pl.ds(g * QT, QT)
        qg = q_ref[0, qs, :]
        qb = qg.astype(jnp.bfloat16)
        qk = jnp.dot(qb, kb, preferred_element_type=jnp.float32)
        q2 = jnp.sum(qg * qg, axis=1)
        d2 = (q2[:, None] + k2[None, :]) - 2.0 * qk
        bits = lax.bitcast_convert_type(d2, jnp.int32)
        sortable = jnp.where(bits >= 0, bits, jnp.int32(INT_MIN) - bits)
        lane = lax.broadcasted_iota(jnp.int32, (QT, N), 1)
        packed = (sortable & jnp.int32(MASK_VAL)) | lane
        pk_ref[qs, :] = packed
        cm = packed[:, :128]
        for v in range(1, NV):
            cm = jnp.minimum(cm, packed[:, v * 128:(v + 1) * 128])
        colmins.append(cm)

    colmin_all = jnp.concatenate(colmins, axis=0)
    for h in range(QTOT // SUB):
        r0 = h * SUB
        colsort = _bitonic128(colmin_all[r0:r0 + SUB])
        idx_cols = colsort[:, :K_NN] & 127
        pk_h = pk_ref[pl.ds(r0, SUB), :]
        cand = jnp.concatenate(
            [jnp.take_along_axis(pk_h[:, v * 128:(v + 1) * 128], idx_cols,
                                 axis=1) for v in range(NV)], axis=1)
        ccm = cand[:, :128]
        for v in range(1, KNV):
            ccm = jnp.minimum(ccm, cand[:, v * 128:(v + 1) * 128])
        clane = _lane128(ccm.shape)
        ccs = _bitonic128((ccm & jnp.int32(~127)) | clane)
        idx_c2 = ccs[:, :K_NN] & 127
        cand2 = jnp.concatenate(
            [jnp.take_along_axis(cand[:, v * 128:(v + 1) * 128], idx_c2,
                                 axis=1) for v in range(KNV)]
            + [jnp.full((SUB, 128 - KNV * K_NN), INT_MAX, jnp.int32)],
            axis=1)
        s = _bitonic128(cand2)
        res = s[:, :K_NN]
        t = res & jnp.int32(MASK_VAL)
        d2t = lax.bitcast_convert_type(
            jnp.where(t >= 0, t, jnp.int32(INT_MIN) - t), jnp.float32)
        dk_ref[0, r0:r0 + SUB] = jnp.exp(
            d2t * neg_inv_ls2[r0:r0 + SUB, None])
        idx_ref[0, r0:r0 + SUB] = res & MASK_IDX


def _topk_call(q_all, kt_all):
    p_count = q_all.shape[0]
    return pl.pallas_call(
        _topk_body,
        grid=(p_count, N // QTOT),
        in_specs=[
            pl.BlockSpec((1, QTOT, 3), lambda p, t: (p, t, 0)),
            pl.BlockSpec((1, 3, N), lambda p, t: (p, 0, 0)),
        ],
        out_specs=[
            pl.BlockSpec((1, QTOT, K_NN), lambda p, t: (p, t, 0)),
            pl.BlockSpec((1, QTOT, K_NN), lambda p, t: (p, t, 0)),
        ],
        out_shape=[
            jax.ShapeDtypeStruct((p_count, N, K_NN), jnp.float32),
            jax.ShapeDtypeStruct((p_count, N, K_NN), jnp.int32),
        ],
        scratch_shapes=[pltpu.VMEM((QTOT, N), jnp.int32)],
    )(q_all, kt_all)


NW = 32
Q_PER_W = (8 * N) // NW
E_PER_W = Q_PER_W * K_NN
LANES = 16
CHUNKS = E_PER_W // LANES


def _sqrt_newton(x):
    b = plsc.bitcast(x, jnp.int32)
    y = plsc.bitcast(jnp.int32(0x1FBD1DF5) + (b >> 1), jnp.float32)
    for _ in range(3):
        y = 0.5 * (y + x / y)
    return y


def _color_body(idx_hbm, dk_hbm, fsrc_hbm, fgt_hbm, out_hbm,
                idx_v, dk_v, fsrc_v, fgt_v, out_v):
    wid = lax.axis_index("s") * 2 + lax.axis_index("c")
    p = wid // 4
    qbase = (wid % 4) * Q_PER_W

    pltpu.sync_copy(idx_hbm.at[p, pl.ds(qbase, Q_PER_W)], idx_v)
    pltpu.sync_copy(dk_hbm.at[p, pl.ds(qbase, Q_PER_W)], dk_v)
    pltpu.sync_copy(fsrc_hbm.at[p], fsrc_v)
    pltpu.sync_copy(fgt_hbm.at[p, pl.ds(qbase, Q_PER_W)], fgt_v)

    lanes = lax.iota(jnp.int32, LANES)

    def body(i, acc):
        base = pl.multiple_of(i * LANES, LANES)
        e = base + lanes
        nn = e // K_NN
        jj = e - nn * K_NN
        idx16 = plsc.load_gather(idx_v, [nn, jj])
        dk16 = plsc.load_gather(dk_v, [nn, jj])
        cd2 = jnp.zeros((LANES,), jnp.float32)
        for c in range(3):
            cc = jnp.full((LANES,), c, jnp.int32)
            hsv = plsc.load_gather(fsrc_v, [idx16, cc])
            fg = plsc.load_gather(fgt_v, [nn, cc])
            d = fg - hsv
            cd2 = cd2 + d * d
        cd = _sqrt_newton(cd2)
        return acc + dk16 * jnp.exp(cd * (-INV_COLOR))

    acc = lax.fori_loop(0, CHUNKS, body, jnp.zeros((LANES,), jnp.float32))
    out_v[...] = acc
    pltpu.sync_copy(out_v, out_hbm.at[wid])


@functools.partial(jax.jit, static_argnums=())
def _color_call(idx3, dk3, fsrc_all, fgt_all):
    mesh = plsc.VectorSubcoreMesh(core_axis_name="c", subcore_axis_name="s")
    return pl.kernel(
        _color_body,
        out_type=jax.ShapeDtypeStruct((NW, LANES), jnp.float32),
        mesh=mesh,
        scratch_types=[
            pltpu.VMEM((Q_PER_W, K_NN), jnp.int32),
            pltpu.VMEM((Q_PER_W, K_NN), jnp.float32),
            pltpu.VMEM((N, 3), jnp.float32),
            pltpu.VMEM((Q_PER_W, 3), jnp.float32),
            pltpu.VMEM((LANES,), jnp.float32),
        ],
        compiler_params=pltpu.CompilerParams(
            use_tc_tiling_on_sc=False, needs_layout_passes=False),
    )(idx3, dk3, fsrc_all, fgt_all)


def kernel(pts_gt_1, feat_gt_1, pts_pred_1, feat_pred_1,
           pts_flowed_1_from_2, feat_flowed_1_from_2,
           pts_gt_2, feat_gt_2, pts_pred_2, feat_pred_2,
           pts_flowed_2_from_1, feat_flowed_2_from_1):
    q_all = jnp.concatenate(
        [pts_gt_1, pts_gt_2, pts_gt_1, pts_gt_2], axis=0)
    k_all = jnp.concatenate(
        [pts_pred_1, pts_pred_2, pts_flowed_1_from_2, pts_flowed_2_from_1],
        axis=0)
    fsrc_all = jnp.concatenate(
        [feat_pred_1, feat_pred_2, feat_flowed_1_from_2,
         feat_flowed_2_from_1], axis=0)[:, :, :3]
    fgt_all = jnp.concatenate(
        [feat_gt_1, feat_gt_2, feat_gt_1, feat_gt_2], axis=0)[:, :, :3]

    kt_all = jnp.transpose(k_all, (0, 2, 1))

    dk, idx = _topk_call(q_all, kt_all)
    partial = _color_call(idx, dk, fsrc_all, fgt_all)
    total = jnp.sum(partial)
    return -(total / jnp.float32(2 * N * K_NN))


# speedup vs baseline: 1.1482x; 1.1407x over previous
import functools

import jax
import jax.numpy as jnp
from jax import lax
from jax.experimental import pallas as pl
from jax.experimental.pallas import tpu as pltpu
from jax.experimental.pallas import tpu_sc as plsc

K_NN = 20
ELL = 0.15
BASEDIST = 10.0
INV_COLOR = 5.0

N = 4096
QT = 8
NQT = N // QT
MASK_IDX = 0xFFF
MASK_VAL = ~0xFFF


G = 32
QTOT = G * QT
NV = N // 128
KNV = K_NN * NV // 128
SUB = 64
INT_MIN = -2147483648
INT_MAX = 0x7FFFFFFF


def _lane128(shape):
    return lax.broadcasted_iota(jnp.int32, shape, 1)


def _xor_perm(x, j, lane):
    return jnp.take_along_axis(x, lane ^ j, axis=1)


def _bitonic128(x):
    lane = _lane128(x.shape)
    for k in (2, 4, 8, 16, 32, 64, 128):
        j = k // 2
        while j >= 1:
            px = _xor_perm(x, j, lane)
            tmin = ((lane & k) == 0) == ((lane & j) == 0)
            x = jnp.where(tmin, jnp.minimum(x, px), jnp.maximum(x, px))
            j //= 2
    return x


def _bitonic_cleanup128(x):
    lane = _lane128(x.shape)
    for j in (64, 32, 16, 8, 4, 2, 1):
        px = _xor_perm(x, j, lane)
        tmin = (lane & j) == 0
        x = jnp.where(tmin, jnp.minimum(x, px), jnp.maximum(x, px))
    return x


def _merge_low128(a, b):
    lane = _lane128(b.shape)
    brev = jnp.take_along_axis(b, 127 - lane, axis=1)
    return _bitonic_cleanup128(jnp.minimum(a, brev))


def _topk_body(q_ref, kt_ref, dk_ref, idx_ref, pk_ref):
    kb = kt_ref[0].astype(jnp.bfloat16)
    k2 = jnp.sum(kt_ref[0] * kt_ref[0], axis=0)

    z = q_ref[0, :, 2]
    ls = jnp.maximum(ELL * (z - BASEDIST) / BASEDIST, ELL)
    neg_inv_ls2 = -1.0 / (ls * ls)

    colmins = []
    for g in range(G):
        qs = pl.ds(g * QT, QT)
        qg = q_ref[0, qs, :]
        qb = qg.astype(jnp.bfloat16)
        qk = jnp.dot(qb, kb, preferred_element_type=jnp.float32)
        q2 = jnp.sum(qg * qg, axis=1)
        d2 = (q2[:, None] + k2[None, :]) - 2.0 * qk
        bits = lax.bitcast_convert_type(d2, jnp.int32)
        sortable = jnp.where(bits >= 0, bits, jnp.int32(INT_MIN) - bits)
        lane = lax.broadcasted_iota(jnp.int32, (QT, N), 1)
        packed = (sortable & jnp.int32(MASK_VAL)) | lane
        pk_ref[qs, :] = packed
        cm = packed[:, :128]
        for v in range(1, NV):
            cm = jnp.minimum(cm, packed[:, v * 128:(v + 1) * 128])
        colmins.append(cm)

    colmin_all = jnp.concatenate(colmins, axis=0)
    for h in range(QTOT // SUB):
        r0 = h * SUB
        colsort = _bitonic128(colmin_all[r0:r0 + SUB])
        idx_cols = colsort[:, :K_NN] & 127
        pk_h = pk_ref[pl.ds(r0, SUB), :]
        cand = jnp.concatenate(
            [jnp.take_along_axis(pk_h[:, v * 128:(v + 1) * 128], idx_cols,
                                 axis=1) for v in range(NV)], axis=1)
        ccm = cand[:, :128]
        for v in range(1, KNV):
            ccm = jnp.minimum(ccm, cand[:, v * 128:(v + 1) * 128])
        clane = _lane128(ccm.shape)
        ccs = _bitonic128((ccm & jnp.int32(~127)) | clane)
        idx_c2 = ccs[:, :K_NN] & 127
        cand2 = jnp.concatenate(
            [jnp.take_along_axis(cand[:, v * 128:(v + 1) * 128], idx_c2,
                                 axis=1) for v in range(KNV)]
            + [jnp.full((SUB, 128 - KNV * K_NN), INT_MAX, jnp.int32)],
            axis=1)
        s = _bitonic128(cand2)
        res = s[:, :K_NN]
        t = res & jnp.int32(MASK_VAL)
        d2t = lax.bitcast_convert_type(
            jnp.where(t >= 0, t, jnp.int32(INT_MIN) - t), jnp.float32)
        dk_ref[0, r0:r0 + SUB] = jnp.exp(
            d2t * neg_inv_ls2[r0:r0 + SUB, None])
        idx_ref[0, r0:r0 + SUB] = res & MASK_IDX


def _topk_call(q_all, kt_all):
    p_count = q_all.shape[0]
    return pl.pallas_call(
        _topk_body,
        grid=(p_count, N // QTOT),
        in_specs=[
            pl.BlockSpec((1, QTOT, 3), lambda p, t: (p, t, 0)),
            pl.BlockSpec((1, 3, N), lambda p, t: (p, 0, 0)),
        ],
        out_specs=[
            pl.BlockSpec((1, QTOT, K_NN), lambda p, t: (p, t, 0)),
            pl.BlockSpec((1, QTOT, K_NN), lambda p, t: (p, t, 0)),
        ],
        out_shape=[
            jax.ShapeDtypeStruct((p_count, N, K_NN), jnp.float32),
            jax.ShapeDtypeStruct((p_count, N, K_NN), jnp.int32),
        ],
        scratch_shapes=[pltpu.VMEM((QTOT, N), jnp.int32)],
    )(q_all, kt_all)


NW = 32
Q_PER_W = (8 * N) // NW
E_PER_W = Q_PER_W * K_NN
LANES = 16
CHUNKS = E_PER_W // LANES


def _sqrt_newton(x):
    b = plsc.bitcast(x, jnp.int32)
    y = plsc.bitcast(jnp.int32(0x1FBD1DF5) + (b >> 1), jnp.float32)
    for _ in range(3):
        y = 0.5 * (y + x / y)
    return y


def _color_body(idx_hbm, dk_hbm, fsrc_hbm, fgt_hbm, out_hbm,
                idx_v, dk_v, fsrc_v, fgt_v, out_v):
    wid = lax.axis_index("s") * 2 + lax.axis_index("c")
    p = wid // 4
    qbase = (wid % 4) * Q_PER_W

    pltpu.sync_copy(idx_hbm.at[p, pl.ds(qbase, Q_PER_W)], idx_v)
    pltpu.sync_copy(dk_hbm.at[p, pl.ds(qbase, Q_PER_W)], dk_v)
    pltpu.sync_copy(fsrc_hbm.at[p], fsrc_v)
    pltpu.sync_copy(fgt_hbm.at[p, pl.ds(qbase, Q_PER_W)], fgt_v)

    lanes = lax.iota(jnp.int32, LANES)

    def body(i, acc):
        base = pl.multiple_of(i * LANES, LANES)
        e = base + lanes
        nn = e // K_NN
        jj = e - nn * K_NN
        idx16 = plsc.load_gather(idx_v, [nn, jj])
        dk16 = plsc.load_gather(dk_v, [nn, jj])
        cd2 = jnp.zeros((LANES,), jnp.float32)
        for c in range(3):
            cc = jnp.full((LANES,), c, jnp.int32)
            hsv = plsc.load_gather(fsrc_v, [idx16, cc])
            fg = plsc.load_gather(fgt_v, [nn, cc])
            d = fg - hsv
            cd2 = cd2 + d * d
        cd = _sqrt_newton(cd2)
        return acc + dk16 * jnp.exp(cd * (-INV_COLOR))

    acc = lax.fori_loop(0, CHUNKS, body, jnp.zeros((LANES,), jnp.float32))
    out_v[...] = acc
    pltpu.sync_copy(out_v, out_hbm.at[wid])


@functools.partial(jax.jit, static_argnums=())
def _color_call(idx3, dk3, fsrc_all, fgt_all):
    mesh = plsc.VectorSubcoreMesh(core_axis_name="c", subcore_axis_name="s")
    return pl.kernel(
        _color_body,
        out_type=jax.ShapeDtypeStruct((NW, LANES), jnp.float32),
        mesh=mesh,
        scratch_types=[
            pltpu.VMEM((Q_PER_W, K_NN), jnp.int32),
            pltpu.VMEM((Q_PER_W, K_NN), jnp.float32),
            pltpu.VMEM((N, 3), jnp.float32),
            pltpu.VMEM((Q_PER_W, 3), jnp.float32),
            pltpu.VMEM((LANES,), jnp.float32),
        ],
        compiler_params=pltpu.CompilerParams(
            use_tc_tiling_on_sc=False, needs_layout_passes=False),
    )(idx3, dk3, fsrc_all, fgt_all)


def kernel(pts_gt_1, feat_gt_1, pts_pred_1, feat_pred_1,
           pts_flowed_1_from_2, feat_flowed_1_from_2,
           pts_gt_2, feat_gt_2, pts_pred_2, feat_pred_2,
           pts_flowed_2_from_1, feat_flowed_2_from_1):
    q_all = jnp.concatenate(
        [pts_gt_1, pts_gt_2, pts_gt_1, pts_gt_2], axis=0)
    k_all = jnp.concatenate(
        [pts_pred_1, pts_pred_2, pts_flowed_1_from_2, pts_flowed_2_from_1],
        axis=0)
    fsrc_all = jnp.concatenate(
        [feat_pred_1, feat_pred_2, feat_flowed_1_from_2,
         feat_flowed_2_from_1], axis=0)[:, :, :3]
    fgt_all = jnp.concatenate(
        [feat_gt_1, feat_gt_2, feat_gt_1, feat_gt_2], axis=0)[:, :, :3]

    kt_all = jnp.transpose(k_all, (0, 2, 1))

    dk, idx = _topk_call(q_all, kt_all)
    partial = _color_call(idx, dk, fsrc_all, fgt_all)
    total = jnp.sum(partial)
    return -(total / jnp.float32(2 * N * K_NN))
